# Initial kernel scaffold; baseline (speedup 1.0000x reference)
import jax
import jax.numpy as jnp
from jax.experimental import pallas as pl


def kernel(X, A, W1, a_src1, a_dst1, W2, a_src2, a_dst2, W3, a_src3, a_dst3):
    raise NotImplementedError("write your pallas kernel here")


# speedup vs baseline: 1.1008x; 1.1008x over previous
import jax
import jax.numpy as jnp
from jax.experimental import pallas as pl

_N = 4096
_BM = 256
_BMX = 512

_HIGH = jax.lax.Precision.HIGHEST


def _phase1_kernel(x_ref, w_ref, asv_ref, adv_ref, hb_ref, asrc_ref, adst_ref):
    h = jnp.dot(x_ref[:], w_ref[:], precision=_HIGH,
                preferred_element_type=jnp.float32)
    hb_ref[:] = h.astype(jnp.bfloat16)
    asrc_ref[:] = jnp.dot(h, asv_ref[:], precision=_HIGH,
                          preferred_element_type=jnp.float32)
    adst_ref[:] = jnp.dot(h, adv_ref[:], precision=_HIGH,
                          preferred_element_type=jnp.float32)


def _phase1(x, w, a_src, a_dst):
    n, fi = x.shape
    fo = w.shape[1]
    hb, asrc, adst = pl.pallas_call(
        _phase1_kernel,
        grid=(n // _BMX,),
        in_specs=[
            pl.BlockSpec((_BMX, fi), lambda i: (i, 0)),
            pl.BlockSpec((fi, fo), lambda i: (0, 0)),
            pl.BlockSpec((fo, 1), lambda i: (0, 0)),
            pl.BlockSpec((fo, 1), lambda i: (0, 0)),
        ],
        out_specs=[
            pl.BlockSpec((_BMX, fo), lambda i: (i, 0)),
            pl.BlockSpec((_BMX, 1), lambda i: (i, 0)),
            pl.BlockSpec((_BMX, 1), lambda i: (i, 0)),
        ],
        out_shape=[
            jax.ShapeDtypeStruct((n, fo), jnp.bfloat16),
            jax.ShapeDtypeStruct((n, 1), jnp.float32),
            jax.ShapeDtypeStruct((n, 1), jnp.float32),
        ],
    )(x, w, a_src.reshape(fo, 1), a_dst.reshape(fo, 1))
    return hb, asrc, adst.reshape(1, n)


def _attn_body(mask, hb_ref, asrc_ref, adst_ref, out_ref):
    i = pl.program_id(0)
    s = asrc_ref[pl.ds(i * _BM, _BM), :] + adst_ref[:]
    s = jnp.maximum(s, 0.2 * s)
    s = jnp.where(mask, s, -1e9)
    p = jnp.exp(s)
    denom = jnp.sum(p, axis=1, keepdims=True)
    acc = jax.lax.dot_general(
        p.astype(jnp.bfloat16), hb_ref[:],
        (((1,), (0,)), ((), ())),
        preferred_element_type=jnp.float32)
    o = acc / denom
    o = o / (jnp.sqrt(jnp.sum(o * o, axis=1, keepdims=True)) + 1e-12)
    out_ref[:] = o


def _attn_kernel_l1(a_ref, hb_ref, asrc_ref, adst_ref, out_ref, mask_ref):
    mask = a_ref[:] > 0.0
    mask_ref[:] = mask.astype(jnp.int8)
    _attn_body(mask, hb_ref, asrc_ref, adst_ref, out_ref)


def _attn_kernel(m_ref, hb_ref, asrc_ref, adst_ref, out_ref):
    _attn_body(m_ref[:] != 0, hb_ref, asrc_ref, adst_ref, out_ref)


def _phase2(mask_src, hb, asrc, adst, first):
    n, fo = hb.shape
    kern = _attn_kernel_l1 if first else _attn_kernel
    out_shape = [jax.ShapeDtypeStruct((n, fo), jnp.float32)]
    out_specs = [pl.BlockSpec((_BM, fo), lambda i: (i, 0))]
    if first:
        out_shape.append(jax.ShapeDtypeStruct((n, n), jnp.int8))
        out_specs.append(pl.BlockSpec((_BM, n), lambda i: (i, 0)))
    res = pl.pallas_call(
        kern,
        grid=(n // _BM,),
        in_specs=[
            pl.BlockSpec((_BM, n), lambda i: (i, 0)),
            pl.BlockSpec((n, fo), lambda i: (0, 0)),
            pl.BlockSpec((n, 1), lambda i: (0, 0)),
            pl.BlockSpec((1, n), lambda i: (0, 0)),
        ],
        out_specs=out_specs,
        out_shape=out_shape,
    )(mask_src, hb, asrc, adst)
    return res if first else (res[0], None)


def kernel(X, A, W1, a_src1, a_dst1, W2, a_src2, a_dst2, W3, a_src3, a_dst3):
    hb, asrc, adst = _phase1(X, W1, a_src1, a_dst1)
    out1, mask = _phase2(A, hb, asrc, adst, first=True)
    hb, asrc, adst = _phase1(out1, W2, a_src2, a_dst2)
    out2, _ = _phase2(mask, hb, asrc, adst, first=False)
    hb, asrc, adst = _phase1(out2, W3, a_src3, a_dst3)
    out3, _ = _phase2(mask, hb, asrc, adst, first=False)
    return (out3, A)


# speedup vs baseline: 1.2172x; 1.1057x over previous
import jax
import jax.numpy as jnp
from jax.experimental import pallas as pl

_N = 4096
_BM = 256
_BMX = 512
_LOG2E = 1.4426950408889634
_NEG = -1e9

_HIGH = jax.lax.Precision.HIGHEST


def _proj(x, w_ref, as_ref, ad_ref):
    w = w_ref[:]
    hb = jnp.dot(x.astype(jnp.bfloat16), w.astype(jnp.bfloat16),
                 preferred_element_type=jnp.float32).astype(jnp.bfloat16)
    was = jnp.dot(w, as_ref[:], precision=_HIGH,
                  preferred_element_type=jnp.float32) * _LOG2E
    wad = jnp.dot(w, ad_ref[:], precision=_HIGH,
                  preferred_element_type=jnp.float32) * _LOG2E
    asrc = jnp.dot(x, was, precision=_HIGH, preferred_element_type=jnp.float32)
    adst = jnp.dot(x, wad, precision=_HIGH, preferred_element_type=jnp.float32)
    return hb, asrc, adst


def _stage_a_kernel(x_ref, w_ref, as_ref, ad_ref, hb_ref, asrc_ref, adst_ref):
    hb_ref[:], asrc_ref[:], adst_ref[:] = _proj(x_ref[:], w_ref, as_ref, ad_ref)


def _attn_core(bias, hb_ref, asrc_ref, adst_ref):
    i = pl.program_id(0)
    s = asrc_ref[pl.ds(i * _BM, _BM), :] + adst_ref[:]
    s = jnp.maximum(s, 0.2 * s) + bias
    p = jnp.exp2(s)
    denom = jnp.sum(p, axis=1, keepdims=True)
    acc = jax.lax.dot_general(
        p.astype(jnp.bfloat16), hb_ref[:],
        (((1,), (0,)), ((), ())),
        preferred_element_type=jnp.float32)
    o = acc / denom
    o = o / (jnp.sqrt(jnp.sum(o * o, axis=1, keepdims=True)) + 1e-12)
    return o


def _stage_b1_kernel(a_ref, hb_ref, asrc_ref, adst_ref, w_ref, as_ref, ad_ref,
                     bias_ref, hb2_ref, asrc2_ref, adst2_ref):
    bias = jnp.where(a_ref[:] > 0.0, 0.0, _NEG)
    bias_ref[:] = bias.astype(jnp.bfloat16)
    o = _attn_core(bias, hb_ref, asrc_ref, adst_ref)
    hb2_ref[:], asrc2_ref[:], adst2_ref[:] = _proj(o, w_ref, as_ref, ad_ref)


def _stage_b2_kernel(bias_ref, hb_ref, asrc_ref, adst_ref, w_ref, as_ref,
                     ad_ref, hb2_ref, asrc2_ref, adst2_ref):
    o = _attn_core(bias_ref[:].astype(jnp.float32), hb_ref, asrc_ref, adst_ref)
    hb2_ref[:], asrc2_ref[:], adst2_ref[:] = _proj(o, w_ref, as_ref, ad_ref)


def _stage_b3_kernel(bias_ref, hb_ref, asrc_ref, adst_ref, out_ref):
    out_ref[:] = _attn_core(bias_ref[:].astype(jnp.float32), hb_ref, asrc_ref,
                            adst_ref)


def _full(shape, dtype=jnp.float32):
    return pl.BlockSpec(shape, lambda i: (0, 0)), jax.ShapeDtypeStruct(
        shape, dtype)


def kernel(X, A, W1, a_src1, a_dst1, W2, a_src2, a_dst2, W3, a_src3, a_dst3):
    n = X.shape[0]
    f1, f2, f3 = W1.shape[1], W2.shape[1], W3.shape[1]

    hb1, asrc1, adst1 = pl.pallas_call(
        _stage_a_kernel,
        grid=(n // _BMX,),
        in_specs=[
            pl.BlockSpec((_BMX, X.shape[1]), lambda i: (i, 0)),
            pl.BlockSpec(W1.shape, lambda i: (0, 0)),
            pl.BlockSpec((f1, 1), lambda i: (0, 0)),
            pl.BlockSpec((f1, 1), lambda i: (0, 0)),
        ],
        out_specs=[
            pl.BlockSpec((_BMX, f1), lambda i: (i, 0)),
            pl.BlockSpec((_BMX, 1), lambda i: (i, 0)),
            pl.BlockSpec((_BMX, 1), lambda i: (i, 0)),
        ],
        out_shape=[
            jax.ShapeDtypeStruct((n, f1), jnp.bfloat16),
            jax.ShapeDtypeStruct((n, 1), jnp.float32),
            jax.ShapeDtypeStruct((n, 1), jnp.float32),
        ],
    )(X, W1, a_src1.reshape(f1, 1), a_dst1.reshape(f1, 1))

    bias, hb2, asrc2, adst2 = pl.pallas_call(
        _stage_b1_kernel,
        grid=(n // _BM,),
        in_specs=[
            pl.BlockSpec((_BM, n), lambda i: (i, 0)),
            pl.BlockSpec((n, f1), lambda i: (0, 0)),
            pl.BlockSpec((n, 1), lambda i: (0, 0)),
            pl.BlockSpec((1, n), lambda i: (0, 0)),
            pl.BlockSpec(W2.shape, lambda i: (0, 0)),
            pl.BlockSpec((f2, 1), lambda i: (0, 0)),
            pl.BlockSpec((f2, 1), lambda i: (0, 0)),
        ],
        out_specs=[
            pl.BlockSpec((_BM, n), lambda i: (i, 0)),
            pl.BlockSpec((_BM, f2), lambda i: (i, 0)),
            pl.BlockSpec((_BM, 1), lambda i: (i, 0)),
            pl.BlockSpec((_BM, 1), lambda i: (i, 0)),
        ],
        out_shape=[
            jax.ShapeDtypeStruct((n, n), jnp.bfloat16),
            jax.ShapeDtypeStruct((n, f2), jnp.bfloat16),
            jax.ShapeDtypeStruct((n, 1), jnp.float32),
            jax.ShapeDtypeStruct((n, 1), jnp.float32),
        ],
    )(A, hb1, asrc1, adst1.reshape(1, n), W2,
      a_src2.reshape(f2, 1), a_dst2.reshape(f2, 1))

    hb3, asrc3, adst3 = pl.pallas_call(
        _stage_b2_kernel,
        grid=(n // _BM,),
        in_specs=[
            pl.BlockSpec((_BM, n), lambda i: (i, 0)),
            pl.BlockSpec((n, f2), lambda i: (0, 0)),
            pl.BlockSpec((n, 1), lambda i: (0, 0)),
            pl.BlockSpec((1, n), lambda i: (0, 0)),
            pl.BlockSpec(W3.shape, lambda i: (0, 0)),
            pl.BlockSpec((f3, 1), lambda i: (0, 0)),
            pl.BlockSpec((f3, 1), lambda i: (0, 0)),
        ],
        out_specs=[
            pl.BlockSpec((_BM, f3), lambda i: (i, 0)),
            pl.BlockSpec((_BM, 1), lambda i: (i, 0)),
            pl.BlockSpec((_BM, 1), lambda i: (i, 0)),
        ],
        out_shape=[
            jax.ShapeDtypeStruct((n, f3), jnp.bfloat16),
            jax.ShapeDtypeStruct((n, 1), jnp.float32),
            jax.ShapeDtypeStruct((n, 1), jnp.float32),
        ],
    )(bias, hb2, asrc2, adst2.reshape(1, n), W3,
      a_src3.reshape(f3, 1), a_dst3.reshape(f3, 1))

    out3 = pl.pallas_call(
        _stage_b3_kernel,
        grid=(n // _BM,),
        in_specs=[
            pl.BlockSpec((_BM, n), lambda i: (i, 0)),
            pl.BlockSpec((n, f3), lambda i: (0, 0)),
            pl.BlockSpec((n, 1), lambda i: (0, 0)),
            pl.BlockSpec((1, n), lambda i: (0, 0)),
        ],
        out_specs=pl.BlockSpec((_BM, f3), lambda i: (i, 0)),
        out_shape=jax.ShapeDtypeStruct((n, f3), jnp.float32),
    )(bias, hb3, asrc3, adst3.reshape(1, n))

    return (out3, A)


# speedup vs baseline: 1.3667x; 1.1228x over previous
import jax
import jax.numpy as jnp
from jax.experimental import pallas as pl

_N = 4096
_BM = 256
_BMX = 512
_LOG2E = 1.4426950408889634
_NEG = -1e9

_HIGH = jax.lax.Precision.HIGHEST


def _proj(x, w_ref, as_ref, ad_ref):
    w = w_ref[:]
    xb = x.astype(jnp.bfloat16)
    hb = jnp.dot(xb, w.astype(jnp.bfloat16),
                 preferred_element_type=jnp.float32).astype(jnp.bfloat16)
    wa = jnp.concatenate([as_ref[:], ad_ref[:]], axis=1)
    wa = jnp.dot(w, wa, precision=_HIGH,
                 preferred_element_type=jnp.float32) * _LOG2E
    al = jnp.dot(xb, wa.astype(jnp.bfloat16),
                 preferred_element_type=jnp.float32)
    return hb, al[:, 0:1], al[:, 1:2]


def _stage_a_kernel(x_ref, w_ref, as_ref, ad_ref, hb_ref, asrc_ref, adst_ref):
    hb_ref[:], asrc_ref[:], adst_ref[:] = _proj(x_ref[:], w_ref, as_ref, ad_ref)


def _attn_core(bias, hb_ref, asrc_ref, adst_ref):
    i = pl.program_id(0)
    s = asrc_ref[pl.ds(i * _BM, _BM), :] + adst_ref[:]
    s = jnp.maximum(s, 0.2 * s) + bias
    p = jnp.exp2(s)
    denom = jnp.sum(p, axis=1, keepdims=True)
    acc = jax.lax.dot_general(
        p.astype(jnp.bfloat16), hb_ref[:],
        (((1,), (0,)), ((), ())),
        preferred_element_type=jnp.float32)
    o = acc / denom
    o = o / (jnp.sqrt(jnp.sum(o * o, axis=1, keepdims=True)) + 1e-12)
    return o


def _stage_b1_kernel(a_ref, hb_ref, asrc_ref, adst_ref, w_ref, as_ref, ad_ref,
                     bias_ref, hb2_ref, asrc2_ref, adst2_ref):
    bias = jnp.where(a_ref[:] > 0.0, 0.0, _NEG)
    bias_ref[:] = bias.astype(jnp.bfloat16)
    o = _attn_core(bias, hb_ref, asrc_ref, adst_ref)
    hb2_ref[:], asrc2_ref[:], adst2_ref[:] = _proj(o, w_ref, as_ref, ad_ref)


def _stage_b2_kernel(bias_ref, hb_ref, asrc_ref, adst_ref, w_ref, as_ref,
                     ad_ref, hb2_ref, asrc2_ref, adst2_ref):
    o = _attn_core(bias_ref[:].astype(jnp.float32), hb_ref, asrc_ref, adst_ref)
    hb2_ref[:], asrc2_ref[:], adst2_ref[:] = _proj(o, w_ref, as_ref, ad_ref)


def _stage_b3_kernel(bias_ref, hb_ref, asrc_ref, adst_ref, out_ref):
    out_ref[:] = _attn_core(bias_ref[:].astype(jnp.float32), hb_ref, asrc_ref,
                            adst_ref)


def _full(shape, dtype=jnp.float32):
    return pl.BlockSpec(shape, lambda i: (0, 0)), jax.ShapeDtypeStruct(
        shape, dtype)


def kernel(X, A, W1, a_src1, a_dst1, W2, a_src2, a_dst2, W3, a_src3, a_dst3):
    n = X.shape[0]
    f1, f2, f3 = W1.shape[1], W2.shape[1], W3.shape[1]

    hb1, asrc1, adst1 = pl.pallas_call(
        _stage_a_kernel,
        grid=(n // _BMX,),
        in_specs=[
            pl.BlockSpec((_BMX, X.shape[1]), lambda i: (i, 0)),
            pl.BlockSpec(W1.shape, lambda i: (0, 0)),
            pl.BlockSpec((f1, 1), lambda i: (0, 0)),
            pl.BlockSpec((f1, 1), lambda i: (0, 0)),
        ],
        out_specs=[
            pl.BlockSpec((_BMX, f1), lambda i: (i, 0)),
            pl.BlockSpec((_BMX, 1), lambda i: (i, 0)),
            pl.BlockSpec((_BMX, 1), lambda i: (i, 0)),
        ],
        out_shape=[
            jax.ShapeDtypeStruct((n, f1), jnp.bfloat16),
            jax.ShapeDtypeStruct((n, 1), jnp.float32),
            jax.ShapeDtypeStruct((n, 1), jnp.float32),
        ],
    )(X, W1, a_src1.reshape(f1, 1), a_dst1.reshape(f1, 1))

    bias, hb2, asrc2, adst2 = pl.pallas_call(
        _stage_b1_kernel,
        grid=(n // _BM,),
        in_specs=[
            pl.BlockSpec((_BM, n), lambda i: (i, 0)),
            pl.BlockSpec((n, f1), lambda i: (0, 0)),
            pl.BlockSpec((n, 1), lambda i: (0, 0)),
            pl.BlockSpec((1, n), lambda i: (0, 0)),
            pl.BlockSpec(W2.shape, lambda i: (0, 0)),
            pl.BlockSpec((f2, 1), lambda i: (0, 0)),
            pl.BlockSpec((f2, 1), lambda i: (0, 0)),
        ],
        out_specs=[
            pl.BlockSpec((_BM, n), lambda i: (i, 0)),
            pl.BlockSpec((_BM, f2), lambda i: (i, 0)),
            pl.BlockSpec((_BM, 1), lambda i: (i, 0)),
            pl.BlockSpec((_BM, 1), lambda i: (i, 0)),
        ],
        out_shape=[
            jax.ShapeDtypeStruct((n, n), jnp.bfloat16),
            jax.ShapeDtypeStruct((n, f2), jnp.bfloat16),
            jax.ShapeDtypeStruct((n, 1), jnp.float32),
            jax.ShapeDtypeStruct((n, 1), jnp.float32),
        ],
    )(A, hb1, asrc1, adst1.reshape(1, n), W2,
      a_src2.reshape(f2, 1), a_dst2.reshape(f2, 1))

    hb3, asrc3, adst3 = pl.pallas_call(
        _stage_b2_kernel,
        grid=(n // _BM,),
        in_specs=[
            pl.BlockSpec((_BM, n), lambda i: (i, 0)),
            pl.BlockSpec((n, f2), lambda i: (0, 0)),
            pl.BlockSpec((n, 1), lambda i: (0, 0)),
            pl.BlockSpec((1, n), lambda i: (0, 0)),
            pl.BlockSpec(W3.shape, lambda i: (0, 0)),
            pl.BlockSpec((f3, 1), lambda i: (0, 0)),
            pl.BlockSpec((f3, 1), lambda i: (0, 0)),
        ],
        out_specs=[
            pl.BlockSpec((_BM, f3), lambda i: (i, 0)),
            pl.BlockSpec((_BM, 1), lambda i: (i, 0)),
            pl.BlockSpec((_BM, 1), lambda i: (i, 0)),
        ],
        out_shape=[
            jax.ShapeDtypeStruct((n, f3), jnp.bfloat16),
            jax.ShapeDtypeStruct((n, 1), jnp.float32),
            jax.ShapeDtypeStruct((n, 1), jnp.float32),
        ],
    )(bias, hb2, asrc2, adst2.reshape(1, n), W3,
      a_src3.reshape(f3, 1), a_dst3.reshape(f3, 1))

    out3 = pl.pallas_call(
        _stage_b3_kernel,
        grid=(n // _BM,),
        in_specs=[
            pl.BlockSpec((_BM, n), lambda i: (i, 0)),
            pl.BlockSpec((n, f3), lambda i: (0, 0)),
            pl.BlockSpec((n, 1), lambda i: (0, 0)),
            pl.BlockSpec((1, n), lambda i: (0, 0)),
        ],
        out_specs=pl.BlockSpec((_BM, f3), lambda i: (i, 0)),
        out_shape=jax.ShapeDtypeStruct((n, f3), jnp.float32),
    )(bias, hb3, asrc3, adst3.reshape(1, n))

    return (out3, A)


# speedup vs baseline: 1.5324x; 1.1213x over previous
import jax
import jax.numpy as jnp
from jax.experimental import pallas as pl

_N = 4096
_BM = 256
_BMX = 512
_LOG2E = 1.4426950408889634
_NEG = -1e9

_HIGH = jax.lax.Precision.HIGHEST


def _proj(x, w_ref, as_ref, ad_ref):
    w = w_ref[:]
    xb = x.astype(jnp.bfloat16)
    h = jnp.dot(xb, w.astype(jnp.bfloat16),
                preferred_element_type=jnp.float32)
    bm = h.shape[0]
    ones = (jax.lax.broadcasted_iota(jnp.int32, (bm, 128), 1) == 0)
    haug = jnp.concatenate([h.astype(jnp.bfloat16),
                            ones.astype(jnp.bfloat16)], axis=1)
    wa = jnp.concatenate([as_ref[:], ad_ref[:]], axis=1)
    wa = jnp.dot(w, wa, precision=_HIGH,
                 preferred_element_type=jnp.float32) * _LOG2E
    al = jnp.dot(xb, wa.astype(jnp.bfloat16),
                 preferred_element_type=jnp.float32)
    al = al.astype(jnp.bfloat16)
    return haug, al[:, 0:1], al[:, 1:2]


def _stage_a_kernel(x_ref, w_ref, as_ref, ad_ref, hb_ref, asrc_ref, adst_ref):
    hb_ref[:], asrc_ref[:], adst_ref[:] = _proj(x_ref[:], w_ref, as_ref, ad_ref)


def _attn_core(bias, hb_ref, asrc_ref, adst_ref):
    i = pl.program_id(0)
    f = hb_ref.shape[1] - 128
    s = asrc_ref[pl.ds(i * _BM, _BM), :] + adst_ref[:]
    s = jnp.maximum(s, jnp.bfloat16(0.2) * s) + bias
    p = jnp.exp2(s)
    acc = jax.lax.dot_general(
        p, hb_ref[:],
        (((1,), (0,)), ((), ())),
        preferred_element_type=jnp.float32)
    o = acc[:, :f] / acc[:, f:f + 1]
    o = o / (jnp.sqrt(jnp.sum(o * o, axis=1, keepdims=True)) + 1e-12)
    return o


def _stage_b1_kernel(a_ref, hb_ref, asrc_ref, adst_ref, w_ref, as_ref, ad_ref,
                     bias_ref, hb2_ref, asrc2_ref, adst2_ref):
    bias = jnp.where(a_ref[:] > 0.0, 0.0, _NEG).astype(jnp.bfloat16)
    bias_ref[:] = bias
    o = _attn_core(bias, hb_ref, asrc_ref, adst_ref)
    hb2_ref[:], asrc2_ref[:], adst2_ref[:] = _proj(o, w_ref, as_ref, ad_ref)


def _stage_b2_kernel(bias_ref, hb_ref, asrc_ref, adst_ref, w_ref, as_ref,
                     ad_ref, hb2_ref, asrc2_ref, adst2_ref):
    o = _attn_core(bias_ref[:], hb_ref, asrc_ref, adst_ref)
    hb2_ref[:], asrc2_ref[:], adst2_ref[:] = _proj(o, w_ref, as_ref, ad_ref)


def _stage_b3_kernel(bias_ref, hb_ref, asrc_ref, adst_ref, out_ref):
    out_ref[:] = _attn_core(bias_ref[:], hb_ref, asrc_ref, adst_ref)


def kernel(X, A, W1, a_src1, a_dst1, W2, a_src2, a_dst2, W3, a_src3, a_dst3):
    n = X.shape[0]
    f1, f2, f3 = W1.shape[1], W2.shape[1], W3.shape[1]
    g1, g2, g3 = f1 + 128, f2 + 128, f3 + 128

    hb1, asrc1, adst1 = pl.pallas_call(
        _stage_a_kernel,
        grid=(n // _BMX,),
        in_specs=[
            pl.BlockSpec((_BMX, X.shape[1]), lambda i: (i, 0)),
            pl.BlockSpec(W1.shape, lambda i: (0, 0)),
            pl.BlockSpec((f1, 1), lambda i: (0, 0)),
            pl.BlockSpec((f1, 1), lambda i: (0, 0)),
        ],
        out_specs=[
            pl.BlockSpec((_BMX, g1), lambda i: (i, 0)),
            pl.BlockSpec((_BMX, 1), lambda i: (i, 0)),
            pl.BlockSpec((_BMX, 1), lambda i: (i, 0)),
        ],
        out_shape=[
            jax.ShapeDtypeStruct((n, g1), jnp.bfloat16),
            jax.ShapeDtypeStruct((n, 1), jnp.bfloat16),
            jax.ShapeDtypeStruct((n, 1), jnp.bfloat16),
        ],
    )(X, W1, a_src1.reshape(f1, 1), a_dst1.reshape(f1, 1))

    bias, hb2, asrc2, adst2 = pl.pallas_call(
        _stage_b1_kernel,
        grid=(n // _BM,),
        in_specs=[
            pl.BlockSpec((_BM, n), lambda i: (i, 0)),
            pl.BlockSpec((n, g1), lambda i: (0, 0)),
            pl.BlockSpec((n, 1), lambda i: (0, 0)),
            pl.BlockSpec((1, n), lambda i: (0, 0)),
            pl.BlockSpec(W2.shape, lambda i: (0, 0)),
            pl.BlockSpec((f2, 1), lambda i: (0, 0)),
            pl.BlockSpec((f2, 1), lambda i: (0, 0)),
        ],
        out_specs=[
            pl.BlockSpec((_BM, n), lambda i: (i, 0)),
            pl.BlockSpec((_BM, g2), lambda i: (i, 0)),
            pl.BlockSpec((_BM, 1), lambda i: (i, 0)),
            pl.BlockSpec((_BM, 1), lambda i: (i, 0)),
        ],
        out_shape=[
            jax.ShapeDtypeStruct((n, n), jnp.bfloat16),
            jax.ShapeDtypeStruct((n, g2), jnp.bfloat16),
            jax.ShapeDtypeStruct((n, 1), jnp.bfloat16),
            jax.ShapeDtypeStruct((n, 1), jnp.bfloat16),
        ],
    )(A, hb1, asrc1, adst1.reshape(1, n), W2,
      a_src2.reshape(f2, 1), a_dst2.reshape(f2, 1))

    hb3, asrc3, adst3 = pl.pallas_call(
        _stage_b2_kernel,
        grid=(n // _BM,),
        in_specs=[
            pl.BlockSpec((_BM, n), lambda i: (i, 0)),
            pl.BlockSpec((n, g2), lambda i: (0, 0)),
            pl.BlockSpec((n, 1), lambda i: (0, 0)),
            pl.BlockSpec((1, n), lambda i: (0, 0)),
            pl.BlockSpec(W3.shape, lambda i: (0, 0)),
            pl.BlockSpec((f3, 1), lambda i: (0, 0)),
            pl.BlockSpec((f3, 1), lambda i: (0, 0)),
        ],
        out_specs=[
            pl.BlockSpec((_BM, g3), lambda i: (i, 0)),
            pl.BlockSpec((_BM, 1), lambda i: (i, 0)),
            pl.BlockSpec((_BM, 1), lambda i: (i, 0)),
        ],
        out_shape=[
            jax.ShapeDtypeStruct((n, g3), jnp.bfloat16),
            jax.ShapeDtypeStruct((n, 1), jnp.bfloat16),
            jax.ShapeDtypeStruct((n, 1), jnp.bfloat16),
        ],
    )(bias, hb2, asrc2, adst2.reshape(1, n), W3,
      a_src3.reshape(f3, 1), a_dst3.reshape(f3, 1))

    out3 = pl.pallas_call(
        _stage_b3_kernel,
        grid=(n // _BM,),
        in_specs=[
            pl.BlockSpec((_BM, n), lambda i: (i, 0)),
            pl.BlockSpec((n, g3), lambda i: (0, 0)),
            pl.BlockSpec((n, 1), lambda i: (0, 0)),
            pl.BlockSpec((1, n), lambda i: (0, 0)),
        ],
        out_specs=pl.BlockSpec((_BM, f3), lambda i: (i, 0)),
        out_shape=jax.ShapeDtypeStruct((n, f3), jnp.float32),
    )(bias, hb3, asrc3, adst3.reshape(1, n))

    return (out3, A)


# speedup vs baseline: 1.8302x; 1.1944x over previous
import jax
import jax.numpy as jnp
from jax.experimental import pallas as pl
from jax.experimental.pallas import tpu as pltpu

_N = 4096
_BM = 256
_BMX = 512
_NB = _N // _BM
_LOG2E = 1.4426950408889634
_NEG = -1e9

_HIGH = jax.lax.Precision.HIGHEST


def _ones_col(bm):
    return (jax.lax.broadcasted_iota(jnp.int32, (bm, 128), 1) == 0)


def _wa2(w_ref, as_ref, ad_ref):
    wa = jnp.concatenate([as_ref[:], ad_ref[:]], axis=1)
    wa = jnp.dot(w_ref[:], wa, precision=_HIGH,
                 preferred_element_type=jnp.float32) * _LOG2E
    return wa.astype(jnp.bfloat16)


def _stage_a_kernel(x_ref, w_ref, as_ref, ad_ref, hb_ref, asrc_ref, adst_ref):
    xb = x_ref[:].astype(jnp.bfloat16)
    h = jnp.dot(xb, w_ref[:].astype(jnp.bfloat16),
                preferred_element_type=jnp.float32)
    hb_ref[:] = jnp.concatenate(
        [h.astype(jnp.bfloat16), _ones_col(h.shape[0]).astype(jnp.bfloat16)],
        axis=1)
    al = jnp.dot(xb, _wa2(w_ref, as_ref, ad_ref),
                 preferred_element_type=jnp.float32).astype(jnp.bfloat16)
    asrc_ref[:] = al[:, 0:1]
    adst_ref[:] = al[:, 1:2]


def _attn(i, bias, hb, asrc_ref, adst_row):
    f = hb.shape[1] - 128
    s = asrc_ref[pl.ds(i * _BM, _BM), :] + adst_row
    s = jnp.maximum(s, jnp.bfloat16(0.2) * s) + bias
    p = jnp.exp2(s)
    acc = jax.lax.dot_general(
        p, hb, (((1,), (0,)), ((), ())),
        preferred_element_type=jnp.float32)
    o = acc[:, :f] / acc[:, f:f + 1]
    o = o / (jnp.sqrt(jnp.sum(o * o, axis=1, keepdims=True)) + 1e-12)
    return o


def _proj_block(o, w_ref, hb_sc, i):
    h = jnp.dot(o.astype(jnp.bfloat16), w_ref[:].astype(jnp.bfloat16),
                preferred_element_type=jnp.float32)
    hb_sc[pl.ds(i * _BM, _BM), :] = jnp.concatenate(
        [h.astype(jnp.bfloat16), _ones_col(_BM).astype(jnp.bfloat16)], axis=1)


def _alphas(hb_sc, f, as_ref, ad_ref, asrc_sc, adst_sc):
    wa = jnp.concatenate([as_ref[:], ad_ref[:]], axis=1) * _LOG2E
    al = jnp.dot(hb_sc[:, :f], wa.astype(jnp.bfloat16),
                 preferred_element_type=jnp.float32).astype(jnp.bfloat16)
    asrc_sc[:, :] = al[:, 0:1]
    adst_sc[:, :] = al[:, 1:2].reshape(1, _N)


def _stage_b_kernel(a_ref, hb1_ref, asrc1_ref, adst1_ref,
                    w2_ref, as2_ref, ad2_ref, w3_ref, as3_ref, ad3_ref,
                    out_ref,
                    bias_sc, hb2_sc, hb3_sc,
                    asrc2_sc, adst2_sc, asrc3_sc, adst3_sc):
    p = pl.program_id(0)

    @pl.when(p < _NB)
    def _layer1():
        i = p
        bias = jnp.where(a_ref[:] > 0.0, 0.0, _NEG).astype(jnp.bfloat16)
        bias_sc[pl.ds(i * _BM, _BM), :] = bias
        o = _attn(i, bias, hb1_ref[:], asrc1_ref, adst1_ref[:])
        _proj_block(o, w2_ref, hb2_sc, i)

    @pl.when(p == _NB)
    def _alpha2():
        _alphas(hb2_sc, w2_ref.shape[1], as2_ref, ad2_ref,
                asrc2_sc, adst2_sc)

    @pl.when((p > _NB) & (p < 2 * _NB + 1))
    def _layer2():
        i = p - _NB - 1
        bias = bias_sc[pl.ds(i * _BM, _BM), :]
        o = _attn(i, bias, hb2_sc[:, :], asrc2_sc, adst2_sc[:, :])
        _proj_block(o, w3_ref, hb3_sc, i)

    @pl.when(p == 2 * _NB + 1)
    def _alpha3():
        _alphas(hb3_sc, w3_ref.shape[1], as3_ref, ad3_ref,
                asrc3_sc, adst3_sc)

    @pl.when(p > 2 * _NB + 1)
    def _layer3():
        i = p - 2 * _NB - 2
        bias = bias_sc[pl.ds(i * _BM, _BM), :]
        out_ref[:, :] = _attn(i, bias, hb3_sc[:, :], asrc3_sc, adst3_sc[:, :])


def kernel(X, A, W1, a_src1, a_dst1, W2, a_src2, a_dst2, W3, a_src3, a_dst3):
    n = X.shape[0]
    f1, f2, f3 = W1.shape[1], W2.shape[1], W3.shape[1]
    g1, g2, g3 = f1 + 128, f2 + 128, f3 + 128

    hb1, asrc1, adst1 = pl.pallas_call(
        _stage_a_kernel,
        grid=(n // _BMX,),
        in_specs=[
            pl.BlockSpec((_BMX, X.shape[1]), lambda i: (i, 0)),
            pl.BlockSpec(W1.shape, lambda i: (0, 0)),
            pl.BlockSpec((f1, 1), lambda i: (0, 0)),
            pl.BlockSpec((f1, 1), lambda i: (0, 0)),
        ],
        out_specs=[
            pl.BlockSpec((_BMX, g1), lambda i: (i, 0)),
            pl.BlockSpec((_BMX, 1), lambda i: (i, 0)),
            pl.BlockSpec((_BMX, 1), lambda i: (i, 0)),
        ],
        out_shape=[
            jax.ShapeDtypeStruct((n, g1), jnp.bfloat16),
            jax.ShapeDtypeStruct((n, 1), jnp.bfloat16),
            jax.ShapeDtypeStruct((n, 1), jnp.bfloat16),
        ],
    )(X, W1, a_src1.reshape(f1, 1), a_dst1.reshape(f1, 1))

    nsteps = 3 * _NB + 2
    out3 = pl.pallas_call(
        _stage_b_kernel,
        grid=(nsteps,),
        in_specs=[
            pl.BlockSpec((_BM, n), lambda p: (jnp.minimum(p, _NB - 1), 0)),
            pl.BlockSpec((n, g1), lambda p: (0, 0)),
            pl.BlockSpec((n, 1), lambda p: (0, 0)),
            pl.BlockSpec((1, n), lambda p: (0, 0)),
            pl.BlockSpec(W2.shape, lambda p: (0, 0)),
            pl.BlockSpec((f2, 1), lambda p: (0, 0)),
            pl.BlockSpec((f2, 1), lambda p: (0, 0)),
            pl.BlockSpec(W3.shape, lambda p: (0, 0)),
            pl.BlockSpec((f3, 1), lambda p: (0, 0)),
            pl.BlockSpec((f3, 1), lambda p: (0, 0)),
        ],
        out_specs=pl.BlockSpec(
            (_BM, f3),
            lambda p: (jnp.clip(p - 2 * _NB - 2, 0, _NB - 1), 0)),
        out_shape=jax.ShapeDtypeStruct((n, f3), jnp.float32),
        scratch_shapes=[
            pltpu.VMEM((n, n), jnp.bfloat16),
            pltpu.VMEM((n, g2), jnp.bfloat16),
            pltpu.VMEM((n, g3), jnp.bfloat16),
            pltpu.VMEM((n, 1), jnp.bfloat16),
            pltpu.VMEM((1, n), jnp.bfloat16),
            pltpu.VMEM((n, 1), jnp.bfloat16),
            pltpu.VMEM((1, n), jnp.bfloat16),
        ],
    )(A, hb1, asrc1, adst1.reshape(1, n), W2,
      a_src2.reshape(f2, 1), a_dst2.reshape(f2, 1), W3,
      a_src3.reshape(f3, 1), a_dst3.reshape(f3, 1))

    return (out3, A)


# speedup vs baseline: 1.9120x; 1.0447x over previous
import jax
import jax.numpy as jnp
from jax.experimental import pallas as pl
from jax.experimental.pallas import tpu as pltpu

_N = 4096
_BM = 256
_BMX = 512
_NB = _N // _BM
_LOG2E = 1.4426950408889634
_NEG = -1e9

_HIGH = jax.lax.Precision.HIGHEST


def _wa2(w_ref, as_ref, ad_ref):
    wa = jnp.concatenate([as_ref[:], ad_ref[:]], axis=1)
    wa = jnp.dot(w_ref[:], wa, precision=_HIGH,
                 preferred_element_type=jnp.float32) * _LOG2E
    return wa.astype(jnp.bfloat16)


def _stage_a_kernel(x_ref, w_ref, as_ref, ad_ref, hb_ref, asrc_ref, adst_ref):
    xb = x_ref[:].astype(jnp.bfloat16)
    h = jnp.dot(xb, w_ref[:].astype(jnp.bfloat16),
                preferred_element_type=jnp.float32)
    hb_ref[:] = h.astype(jnp.bfloat16)
    al = jnp.dot(xb, _wa2(w_ref, as_ref, ad_ref),
                 preferred_element_type=jnp.float32).astype(jnp.bfloat16)
    asrc_ref[:] = al[:, 0:1]
    adst_ref[:] = al[:, 1:2]


def _attn(i, bias, hb, asrc_ref, adst_row):
    s = asrc_ref[pl.ds(i * _BM, _BM), :] + adst_row
    s = jnp.maximum(s, jnp.bfloat16(0.2) * s) + bias
    p = jnp.exp2(s)
    acc = jax.lax.dot_general(
        p, hb, (((1,), (0,)), ((), ())),
        preferred_element_type=jnp.float32)
    return acc * jax.lax.rsqrt(jnp.sum(acc * acc, axis=1, keepdims=True))


def _proj_block(o, w_ref, hb_sc, i):
    h = jnp.dot(o.astype(jnp.bfloat16), w_ref[:],
                preferred_element_type=jnp.float32)
    hb_sc[pl.ds(i * _BM, _BM), :] = h.astype(jnp.bfloat16)


def _alphas(hb_sc, as_ref, ad_ref, asrc_sc, adst_sc):
    wa = jnp.concatenate([as_ref[:], ad_ref[:]], axis=1) * _LOG2E
    al = jnp.dot(hb_sc[:, :], wa.astype(jnp.bfloat16),
                 preferred_element_type=jnp.float32).astype(jnp.bfloat16)
    asrc_sc[:, :] = al[:, 0:1]
    adst_sc[:, :] = al[:, 1:2].reshape(1, _N)


def _stage_b_kernel(a_ref, hb1_ref, asrc1_ref, adst1_ref,
                    w2_ref, as2_ref, ad2_ref, w3_ref, as3_ref, ad3_ref,
                    out_ref,
                    bias_sc, hb2_sc, hb3_sc,
                    asrc2_sc, adst2_sc, asrc3_sc, adst3_sc):
    p = pl.program_id(0)

    @pl.when(p < _NB)
    def _layer1():
        i = p
        bias = jnp.where(a_ref[:] > 0.0, 0.0, _NEG).astype(jnp.bfloat16)
        bias_sc[pl.ds(i * _BM, _BM), :] = bias
        o = _attn(i, bias, hb1_ref[:], asrc1_ref, adst1_ref[:])
        _proj_block(o, w2_ref, hb2_sc, i)

    @pl.when(p == _NB)
    def _alpha2():
        _alphas(hb2_sc, as2_ref, ad2_ref, asrc2_sc, adst2_sc)

    @pl.when((p > _NB) & (p < 2 * _NB + 1))
    def _layer2():
        i = p - _NB - 1
        bias = bias_sc[pl.ds(i * _BM, _BM), :]
        o = _attn(i, bias, hb2_sc[:, :], asrc2_sc, adst2_sc[:, :])
        _proj_block(o, w3_ref, hb3_sc, i)

    @pl.when(p == 2 * _NB + 1)
    def _alpha3():
        _alphas(hb3_sc, as3_ref, ad3_ref, asrc3_sc, adst3_sc)

    @pl.when(p > 2 * _NB + 1)
    def _layer3():
        i = p - 2 * _NB - 2
        bias = bias_sc[pl.ds(i * _BM, _BM), :]
        out_ref[:, :] = _attn(i, bias, hb3_sc[:, :], asrc3_sc,
                                  adst3_sc[:, :])


def kernel(X, A, W1, a_src1, a_dst1, W2, a_src2, a_dst2, W3, a_src3, a_dst3):
    n = X.shape[0]
    f1, f2, f3 = W1.shape[1], W2.shape[1], W3.shape[1]

    hb1, asrc1, adst1 = pl.pallas_call(
        _stage_a_kernel,
        grid=(n // _BMX,),
        in_specs=[
            pl.BlockSpec((_BMX, X.shape[1]), lambda i: (i, 0)),
            pl.BlockSpec(W1.shape, lambda i: (0, 0)),
            pl.BlockSpec((f1, 1), lambda i: (0, 0)),
            pl.BlockSpec((f1, 1), lambda i: (0, 0)),
        ],
        out_specs=[
            pl.BlockSpec((_BMX, f1), lambda i: (i, 0)),
            pl.BlockSpec((_BMX, 1), lambda i: (i, 0)),
            pl.BlockSpec((_BMX, 1), lambda i: (i, 0)),
        ],
        out_shape=[
            jax.ShapeDtypeStruct((n, f1), jnp.bfloat16),
            jax.ShapeDtypeStruct((n, 1), jnp.bfloat16),
            jax.ShapeDtypeStruct((n, 1), jnp.bfloat16),
        ],
    )(X, W1, a_src1.reshape(f1, 1), a_dst1.reshape(f1, 1))

    nsteps = 3 * _NB + 2
    out3 = pl.pallas_call(
        _stage_b_kernel,
        grid=(nsteps,),
        in_specs=[
            pl.BlockSpec((_BM, n), lambda p: (jnp.minimum(p, _NB - 1), 0)),
            pl.BlockSpec((n, f1), lambda p: (0, 0)),
            pl.BlockSpec((n, 1), lambda p: (0, 0)),
            pl.BlockSpec((1, n), lambda p: (0, 0)),
            pl.BlockSpec(W2.shape, lambda p: (0, 0)),
            pl.BlockSpec((f2, 1), lambda p: (0, 0)),
            pl.BlockSpec((f2, 1), lambda p: (0, 0)),
            pl.BlockSpec(W3.shape, lambda p: (0, 0)),
            pl.BlockSpec((f3, 1), lambda p: (0, 0)),
            pl.BlockSpec((f3, 1), lambda p: (0, 0)),
        ],
        out_specs=pl.BlockSpec(
            (_BM, f3),
            lambda p: (jnp.clip(p - 2 * _NB - 2, 0, _NB - 1), 0)),
        out_shape=jax.ShapeDtypeStruct((n, f3), jnp.float32),
        scratch_shapes=[
            pltpu.VMEM((n, n), jnp.bfloat16),
            pltpu.VMEM((n, f2), jnp.bfloat16),
            pltpu.VMEM((n, f3), jnp.bfloat16),
            pltpu.VMEM((n, 1), jnp.bfloat16),
            pltpu.VMEM((1, n), jnp.bfloat16),
            pltpu.VMEM((n, 1), jnp.bfloat16),
            pltpu.VMEM((1, n), jnp.bfloat16),
        ],
    )(A, hb1, asrc1, adst1.reshape(1, n), W2.astype(jnp.bfloat16),
      a_src2.reshape(f2, 1), a_dst2.reshape(f2, 1), W3.astype(jnp.bfloat16),
      a_src3.reshape(f3, 1), a_dst3.reshape(f3, 1))

    return (out3, A)
